# Initial kernel scaffold; baseline (speedup 1.0000x reference)
import jax
import jax.numpy as jnp
from jax.experimental import pallas as pl


def kernel(input_ids, attention_mask, embed_table):
    raise NotImplementedError("write your pallas kernel here")


# speedup vs baseline: 4.3140x; 4.3140x over previous
import functools

import jax
import jax.numpy as jnp
from jax import lax
from jax.experimental import pallas as pl
from jax.experimental.pallas import tpu as pltpu
from jax.experimental.pallas import tpu_sc as plsc

D_MODEL = 256
NUM_WORKERS = 32
CHUNK = 128


def _make_gather(n_tokens: int):
    per_worker = n_tokens // NUM_WORKERS
    n_chunks = per_worker // CHUNK
    mesh = plsc.VectorSubcoreMesh(core_axis_name="c", subcore_axis_name="s")

    @functools.partial(
        pl.kernel,
        mesh=mesh,
        out_type=jax.ShapeDtypeStruct((n_tokens, D_MODEL), jnp.float32),
        scratch_types=[
            pltpu.VMEM((n_chunks, CHUNK), jnp.int32),
            pltpu.VMEM((CHUNK, D_MODEL), jnp.float32),
            pltpu.SemaphoreType.DMA,
        ],
    )
    def gather_kernel(table_hbm, idx_hbm, out_hbm, idx_v, rows_v, sem):
        wid = lax.axis_index("s") * 2 + lax.axis_index("c")
        base = wid * per_worker
        pltpu.sync_copy(idx_hbm.at[wid], idx_v)

        def chunk_body(j, _):
            pltpu.async_copy(table_hbm.at[idx_v.at[j]], rows_v, sem).wait()
            pltpu.sync_copy(rows_v, out_hbm.at[pl.ds(base + j * CHUNK, CHUNK)])
            return 0

        lax.fori_loop(0, n_chunks, chunk_body, 0)

    return gather_kernel


def kernel(input_ids, attention_mask, embed_table):
    batch, seq = input_ids.shape
    n_tokens = batch * seq
    idx = input_ids.reshape(NUM_WORKERS, (n_tokens // NUM_WORKERS) // CHUNK, CHUNK)
    flat = _make_gather(n_tokens)(embed_table, idx)
    emb = flat.reshape(batch, seq, D_MODEL)
    return (emb, input_ids, attention_mask)


# speedup vs baseline: 5.0080x; 1.1609x over previous
import functools

import jax
import jax.numpy as jnp
from jax import lax
from jax.experimental import pallas as pl
from jax.experimental.pallas import tpu as pltpu
from jax.experimental.pallas import tpu_sc as plsc

D_MODEL = 256
NUM_WORKERS = 32
CHUNK = 128


def _make_gather(n_tokens: int):
    per_worker = n_tokens // NUM_WORKERS
    n_chunks = per_worker // CHUNK
    n_pairs = n_chunks // 2
    mesh = plsc.VectorSubcoreMesh(core_axis_name="c", subcore_axis_name="s")

    @functools.partial(
        pl.kernel,
        mesh=mesh,
        out_type=jax.ShapeDtypeStruct((n_tokens, D_MODEL), jnp.float32),
        scratch_types=[
            pltpu.VMEM((n_chunks, CHUNK), jnp.int32),
            pltpu.VMEM((CHUNK, D_MODEL), jnp.float32),
            pltpu.VMEM((CHUNK, D_MODEL), jnp.float32),
            pltpu.SemaphoreType.DMA,
            pltpu.SemaphoreType.DMA,
            pltpu.SemaphoreType.DMA,
            pltpu.SemaphoreType.DMA,
        ],
    )
    def gather_kernel(table_hbm, idx_hbm, out_hbm, idx_v,
                      rows_a, rows_b, gsem_a, gsem_b, ssem_a, ssem_b):
        wid = lax.axis_index("s") * 2 + lax.axis_index("c")
        base = wid * per_worker
        pltpu.sync_copy(idx_hbm.at[wid], idx_v)

        def start_gather(j, buf, sem):
            pltpu.async_copy(table_hbm.at[idx_v.at[j]], buf, sem)

        def wait_gather(buf, sem):
            pltpu.make_async_copy(table_hbm.at[idx_v.at[0]], buf, sem).wait()

        def start_scatter(j, buf, sem):
            pltpu.async_copy(buf, out_hbm.at[pl.ds(base + j * CHUNK, CHUNK)], sem)

        def wait_scatter(buf, sem):
            pltpu.make_async_copy(buf, out_hbm.at[pl.ds(base, CHUNK)], sem).wait()

        start_gather(0, rows_a, gsem_a)

        def body(gg, _):
            j = 2 * gg

            wait_gather(rows_a, gsem_a)

            @pl.when(gg > 0)
            def _():
                wait_scatter(rows_b, ssem_b)

            start_gather(j + 1, rows_b, gsem_b)
            start_scatter(j, rows_a, ssem_a)

            wait_gather(rows_b, gsem_b)
            wait_scatter(rows_a, ssem_a)

            @pl.when(gg < n_pairs - 1)
            def _():
                start_gather(j + 2, rows_a, gsem_a)

            start_scatter(j + 1, rows_b, ssem_b)
            return 0

        lax.fori_loop(0, n_pairs, body, 0)
        wait_scatter(rows_b, ssem_b)

    return gather_kernel


def kernel(input_ids, attention_mask, embed_table):
    batch, seq = input_ids.shape
    n_tokens = batch * seq
    idx = input_ids.reshape(NUM_WORKERS, (n_tokens // NUM_WORKERS) // CHUNK, CHUNK)
    flat = _make_gather(n_tokens)(embed_table, idx)
    emb = flat.reshape(batch, seq, D_MODEL)
    return (emb, input_ids, attention_mask)


# speedup vs baseline: 5.1151x; 1.0214x over previous
import functools

import jax
import jax.numpy as jnp
from jax import lax
from jax.experimental import pallas as pl
from jax.experimental.pallas import tpu as pltpu
from jax.experimental.pallas import tpu_sc as plsc

D_MODEL = 256
NUM_WORKERS = 32
CHUNK = 64
NBUF = 4


def _make_gather(n_tokens: int):
    per_worker = n_tokens // NUM_WORKERS
    n_chunks = per_worker // CHUNK
    n_groups = n_chunks // NBUF
    mesh = plsc.VectorSubcoreMesh(core_axis_name="c", subcore_axis_name="s")

    @functools.partial(
        pl.kernel,
        mesh=mesh,
        out_type=jax.ShapeDtypeStruct((n_tokens, D_MODEL), jnp.float32),
        scratch_types=[
            pltpu.VMEM((n_chunks, CHUNK), jnp.int32),
        ] + [pltpu.VMEM((CHUNK, D_MODEL), jnp.float32)] * NBUF
          + [pltpu.SemaphoreType.DMA] * (2 * NBUF),
    )
    def gather_kernel(table_hbm, idx_hbm, out_hbm, idx_v, *bufs_and_sems):
        bufs = bufs_and_sems[:NBUF]
        gsem = bufs_and_sems[NBUF:2 * NBUF]
        ssem = bufs_and_sems[2 * NBUF:]
        wid = lax.axis_index("s") * 2 + lax.axis_index("c")
        base = wid * per_worker
        pltpu.sync_copy(idx_hbm.at[wid], idx_v)

        def start_gather(j, b):
            pltpu.async_copy(table_hbm.at[idx_v.at[j]], bufs[b], gsem[b])

        def wait_gather(b):
            pltpu.make_async_copy(
                table_hbm.at[idx_v.at[0]], bufs[b], gsem[b]).wait()

        def start_scatter(j, b):
            pltpu.async_copy(
                bufs[b], out_hbm.at[pl.ds(base + j * CHUNK, CHUNK)], ssem[b])

        def wait_scatter(b):
            pltpu.make_async_copy(
                bufs[b], out_hbm.at[pl.ds(base, CHUNK)], ssem[b]).wait()

        start_gather(0, 0)
        start_gather(1, 1)

        def body(g, _):
            j0 = NBUF * g
            for b in range(NBUF):
                j = j0 + b
                wait_gather(b)
                start_scatter(j, b)
                nb = (b + 2) % NBUF
                if b < 2:
                    @pl.when(g > 0)
                    def _():
                        wait_scatter(nb)
                        start_gather(j + 2, nb)

                    @pl.when(g == 0)
                    def _():
                        start_gather(j + 2, nb)
                else:
                    wait_scatter(nb)

                    @pl.when(g < n_groups - 1)
                    def _():
                        start_gather(j + 2, nb)
            return 0

        lax.fori_loop(0, n_groups, body, 0)
        wait_scatter(2)
        wait_scatter(3)

    return gather_kernel


def kernel(input_ids, attention_mask, embed_table):
    batch, seq = input_ids.shape
    n_tokens = batch * seq
    idx = input_ids.reshape(NUM_WORKERS, (n_tokens // NUM_WORKERS) // CHUNK, CHUNK)
    flat = _make_gather(n_tokens)(embed_table, idx)
    emb = flat.reshape(batch, seq, D_MODEL)
    return (emb, input_ids, attention_mask)


# speedup vs baseline: 5.1342x; 1.0037x over previous
import functools

import jax
import jax.numpy as jnp
from jax import lax
from jax.experimental import pallas as pl
from jax.experimental.pallas import tpu as pltpu
from jax.experimental.pallas import tpu_sc as plsc

D_MODEL = 256
NUM_WORKERS = 32
CHUNK = 32
NBUF = 8
GL = 4


def _make_gather(n_tokens: int):
    per_worker = n_tokens // NUM_WORKERS
    n_chunks = per_worker // CHUNK
    n_groups = n_chunks // NBUF
    mesh = plsc.VectorSubcoreMesh(core_axis_name="c", subcore_axis_name="s")

    @functools.partial(
        pl.kernel,
        mesh=mesh,
        out_type=jax.ShapeDtypeStruct((n_tokens, D_MODEL), jnp.float32),
        scratch_types=[
            pltpu.VMEM((n_chunks, CHUNK), jnp.int32),
        ] + [pltpu.VMEM((CHUNK, D_MODEL), jnp.float32)] * NBUF
          + [pltpu.SemaphoreType.DMA] * (2 * NBUF),
    )
    def gather_kernel(table_hbm, idx_hbm, out_hbm, idx_v, *bufs_and_sems):
        bufs = bufs_and_sems[:NBUF]
        gsem = bufs_and_sems[NBUF:2 * NBUF]
        ssem = bufs_and_sems[2 * NBUF:]
        wid = lax.axis_index("s") * 2 + lax.axis_index("c")
        base = wid * per_worker
        pltpu.sync_copy(idx_hbm.at[wid], idx_v)

        def start_gather(j, b):
            pltpu.async_copy(table_hbm.at[idx_v.at[j]], bufs[b], gsem[b])

        def wait_gather(b):
            pltpu.make_async_copy(
                table_hbm.at[idx_v.at[0]], bufs[b], gsem[b]).wait()

        def start_scatter(j, b):
            pltpu.async_copy(
                bufs[b], out_hbm.at[pl.ds(base + j * CHUNK, CHUNK)], ssem[b])

        def wait_scatter(b):
            pltpu.make_async_copy(
                bufs[b], out_hbm.at[pl.ds(base, CHUNK)], ssem[b]).wait()

        for b in range(GL):
            start_gather(b, b)

        def body(g, _):
            j0 = NBUF * g
            for b in range(NBUF):
                j = j0 + b
                wait_gather(b)
                start_scatter(j, b)
                nb = (b + GL) % NBUF
                if b < NBUF - GL:
                    @pl.when(g > 0)
                    def _():
                        wait_scatter(nb)
                        start_gather(j + GL, nb)

                    @pl.when(g == 0)
                    def _():
                        start_gather(j + GL, nb)
                else:
                    wait_scatter(nb)

                    @pl.when(g < n_groups - 1)
                    def _():
                        start_gather(j + GL, nb)
            return 0

        lax.fori_loop(0, n_groups, body, 0)
        for b in range(NBUF - GL, NBUF):
            wait_scatter(b)

    return gather_kernel


def kernel(input_ids, attention_mask, embed_table):
    batch, seq = input_ids.shape
    n_tokens = batch * seq
    idx = input_ids.reshape(NUM_WORKERS, (n_tokens // NUM_WORKERS) // CHUNK, CHUNK)
    flat = _make_gather(n_tokens)(embed_table, idx)
    emb = flat.reshape(batch, seq, D_MODEL)
    return (emb, input_ids, attention_mask)
